# baseline (speedup 1.0000x reference)
import math
import functools

import jax
import jax.numpy as jnp
import numpy as np
from jax import lax
from jax.experimental import pallas as pl
from jax.experimental.pallas import tpu as pltpu
from jax.experimental.pallas import tpu_sc as plsc

BSZ = 4
SEQ = 8192
DIM = 1024
NUM_TOKENS = BSZ * SEQ
NW = 32
CHUNK = NUM_TOKENS // NW
CPR = SEQ // CHUNK
G = 32
NG = CHUNK // G
LANES = 16

_HALF = DIM // 2
_EMB_SCALE = math.log(10000.0) / (_HALF - 1)


def _build_table():
    freqs = np.exp(np.arange(_HALF, dtype=np.float32) * -_EMB_SCALE)
    ang = np.arange(SEQ + 1, dtype=np.float32)[:, None] * freqs[None, :]
    tab = np.concatenate([np.sin(ang), np.cos(ang)], axis=1).astype(np.float32)
    tab[0, :] = 0.0
    return tab


_TABLE = _build_table()


def _sc_body(table_hbm, mask_hbm, out_hbm, row_v, pos_v, rows0, rows1,
             sg0, sg1, so0, so1):
    wid = lax.axis_index("s") * 2 + lax.axis_index("c")
    b = wid // CPR
    c = wid % CPR
    row_base = b * SEQ
    cbase = c * CHUNK
    out_base = row_base + cbase

    pltpu.sync_copy(mask_hbm.at[pl.ds(row_base, SEQ)], row_v)

    def _ofs(i, acc):
        return acc + jnp.sum(row_v[pl.ds(i * LANES, LANES)])

    offset = lax.fori_loop(0, c * (CHUNK // LANES), _ofs, jnp.int32(0))

    carry = offset
    for i in range(CHUNK // LANES):
        v = row_v[pl.ds(cbase + i * LANES, LANES)]
        cum = plsc.cumsum(v) + carry
        carry = carry + jnp.sum(v)
        pos_v[i // 2, pl.ds((i % 2) * LANES, LANES)] = cum * v

    pltpu.async_copy(table_hbm.at[pos_v.at[0]], rows0, sg0)
    pltpu.async_copy(table_hbm.at[pos_v.at[1]], rows1, sg1)

    def _slab(it, _):
        g0 = it * 2
        g1 = g0 + 1
        pltpu.make_async_copy(table_hbm.at[pos_v.at[g0]], rows0, sg0).wait()
        pltpu.async_copy(rows0, out_hbm.at[pl.ds(out_base + g0 * G, G)], so0)
        pltpu.make_async_copy(table_hbm.at[pos_v.at[g1]], rows1, sg1).wait()
        pltpu.async_copy(rows1, out_hbm.at[pl.ds(out_base + g1 * G, G)], so1)

        @pl.when(it + 1 < NG // 2)
        def _():
            pltpu.make_async_copy(rows0, out_hbm.at[pl.ds(out_base, G)], so0).wait()
            pltpu.async_copy(table_hbm.at[pos_v.at[g0 + 2]], rows0, sg0)
            pltpu.make_async_copy(rows1, out_hbm.at[pl.ds(out_base, G)], so1).wait()
            pltpu.async_copy(table_hbm.at[pos_v.at[g1 + 2]], rows1, sg1)

        return 0

    lax.fori_loop(0, NG // 2, _slab, 0)
    pltpu.make_async_copy(rows0, out_hbm.at[pl.ds(out_base, G)], so0).wait()
    pltpu.make_async_copy(rows1, out_hbm.at[pl.ds(out_base, G)], so1).wait()


_sc_kernel = functools.partial(
    pl.kernel,
    out_type=jax.ShapeDtypeStruct((NUM_TOKENS, DIM), jnp.float32),
    mesh=plsc.VectorSubcoreMesh(core_axis_name="c", subcore_axis_name="s"),
    compiler_params=pltpu.CompilerParams(needs_layout_passes=False),
    scratch_types=[
        pltpu.VMEM((SEQ,), jnp.int32),
        pltpu.VMEM((NG, G), jnp.int32),
        pltpu.VMEM((G, DIM), jnp.float32),
        pltpu.VMEM((G, DIM), jnp.float32),
        pltpu.SemaphoreType.DMA,
        pltpu.SemaphoreType.DMA,
        pltpu.SemaphoreType.DMA,
        pltpu.SemaphoreType.DMA,
    ],
)(_sc_body)


@jax.jit
def kernel(pad_mask):
    bsz, seq_len = pad_mask.shape
    mask = jnp.logical_not(pad_mask).astype(jnp.int32).reshape(-1)
    table = jnp.asarray(_TABLE)
    out = _sc_kernel(table, mask)
    return out.reshape(bsz, seq_len, DIM)
